# baseline (speedup 1.0000x reference)
import functools

import jax
import jax.numpy as jnp
from jax import lax
from jax.experimental import pallas as pl
from jax.experimental.pallas import tpu as pltpu
from jax.experimental.pallas import tpu_sc as plsc

_NODES = [10000, 2500, 625, 160, 40]

_NC, _NS, _LANES = 2, 16, 16
_NW = _NC * _NS
_K = 128


def _cdiv(a, b):
    return -(-a // b)


def _segsum_sc(h, src, dst, n_nodes):
    C = h.shape[1]
    E = src.shape[0]
    ZR = max(8, min(32, 8192 // C))
    Npad = _cdiv(n_nodes + 1, _NS * ZR) * (_NS * ZR)
    slab = Npad // _NS
    K = 128 if C <= 256 else 64
    Epad = _cdiv(E, K * _NW * 2) * (K * _NW * 2)
    cpw = Epad // (K * _NW)
    G = min(8, cpw)
    nsup = _cdiv(cpw, G)
    per_tile = 2 * G * K + 2 * K * C + ZR * C
    assert per_tile <= 126000 and _NS * per_tile + Npad * C <= 2000000
    if Epad != E:
        pad = Epad - E
        src = jnp.concatenate([src, jnp.arange(pad, dtype=jnp.int32) % n_nodes])
        dst = jnp.concatenate([
            dst, n_nodes + jnp.arange(pad, dtype=jnp.int32) % (Npad - n_nodes)])
    tail = (nsup * G - cpw) * K
    if tail:
        src = jnp.pad(src, (0, tail))
        dst = jnp.pad(dst, (0, tail))
    src2 = src.reshape(-1, K)
    dst2 = dst.reshape(-1, K)

    mesh = plsc.VectorSubcoreMesh(core_axis_name="c", subcore_axis_name="s")

    @functools.partial(
        pl.kernel, mesh=mesh,
        out_type=jax.ShapeDtypeStruct((_NC, Npad, C), jnp.float32),
        scratch_types=[
            pltpu.VMEM((G, K), jnp.int32),
            pltpu.VMEM((G, K), jnp.int32),
            pltpu.VMEM((2, K, C), jnp.float32),
            pltpu.VMEM((ZR, C), jnp.float32),
            pltpu.VMEM_SHARED((Npad, C), jnp.float32),
            pltpu.SemaphoreType.DMA,
            pltpu.SemaphoreType.DMA,
            pltpu.SemaphoreType.DMA,
        ],
        compiler_params=pltpu.CompilerParams(use_tc_tiling_on_sc=False),
    )
    def k(src_h, dst_h, h_h, out_h, idx_s, idx_d, rows, zrow, acc, si, g0, g1):
        cid = lax.axis_index("c")
        sid = lax.axis_index("s")
        wid = sid * _NC + cid
        c0 = wid * cpw

        zero = jnp.zeros((_LANES,), jnp.float32)
        for r in range(ZR):
            for j in range(C // _LANES):
                zrow[r, pl.ds(j * _LANES, _LANES)] = zero

        @pl.when(sid * slab < n_nodes)
        def _():
            for t in range(slab // ZR):
                pltpu.sync_copy(zrow, acc.at[pl.ds(sid * slab + t * ZR, ZR)])

        plsc.subcore_barrier()

        gsem = (g0, g1)

        def sup(sc, carry):
            base = c0 + sc * G
            ng = jnp.minimum(cpw - sc * G, G)
            cp_s = pltpu.async_copy(src_h.at[pl.ds(base, G)], idx_s, si)
            cp_d = pltpu.async_copy(dst_h.at[pl.ds(base, G)], idx_d, si)
            cp_s.wait()
            cp_d.wait()
            pltpu.async_copy(h_h.at[idx_s.at[0]], rows.at[0], g0)
            for j in range(G):
                b = j & 1

                @pl.when(j + 1 < ng)
                def _(j=j, b=b):
                    pltpu.async_copy(h_h.at[idx_s.at[j + 1]], rows.at[1 - b],
                                     gsem[1 - b])

                @pl.when(j < ng)
                def _(j=j, b=b):
                    pltpu.make_async_copy(h_h.at[idx_s.at[j]], rows.at[b],
                                          gsem[b]).wait()
                    pltpu.sync_copy(rows.at[b], acc.at[idx_d.at[j]], add=True)
            return carry

        lax.fori_loop(0, nsup, sup, 0)
        plsc.subcore_barrier()

        @pl.when(sid * slab < n_nodes)
        def _():
            pltpu.sync_copy(acc.at[pl.ds(sid * slab, slab)],
                            out_h.at[cid, pl.ds(sid * slab, slab)])

    return k(src2, dst2, h)


def _layer_body(n, c_in, aggp_ref, h_ref, wn_ref, ws_ref, g_ref, b_ref, o_ref):
    agg = aggp_ref[0, :n, :c_in] + aggp_ref[1, :n, :c_in]
    z = jnp.dot(agg, wn_ref[...], preferred_element_type=jnp.float32)
    z = z + jnp.dot(h_ref[...], ws_ref[...], preferred_element_type=jnp.float32)
    mu = jnp.mean(z, axis=0, keepdims=True)
    zc = z - mu
    var = jnp.mean(zc * zc, axis=0, keepdims=True)
    zn = zc * lax.rsqrt(var + 1e-5) * g_ref[...] + b_ref[...]
    o_ref[...] = jnp.maximum(zn, 0.0)


def _layer(aggp, h, wn, ws, g, b):
    n, cout = h.shape[0], wn.shape[1]
    return pl.pallas_call(
        functools.partial(_layer_body, n, wn.shape[0]),
        out_shape=jax.ShapeDtypeStruct((n, cout), jnp.float32),
    )(aggp, h, wn, ws, g.reshape(1, -1), b.reshape(1, -1))


def _emb_body(o2_ref, o3_ref, o4_ref, out_ref):
    m2 = jnp.mean(o2_ref[...], axis=0, keepdims=True)
    m3 = jnp.mean(o3_ref[...], axis=0, keepdims=True)
    m4 = jnp.mean(o4_ref[...], axis=0, keepdims=True)
    ss = jnp.sum(m2 * m2) + jnp.sum(m3 * m3) + jnp.sum(m4 * m4)
    inv = 1.0 / jnp.maximum(jnp.sqrt(ss), 1e-12)
    out_ref[0:1, 0:128] = m2 * inv
    out_ref[0:1, 128:384] = m3 * inv
    out_ref[0:1, 384:896] = m4 * inv


def _embed(o2, o3, o4):
    return pl.pallas_call(
        _emb_body,
        out_shape=jax.ShapeDtypeStruct((1, 896), jnp.float32),
    )(o2, o3, o4)


def _pool_sc(h, ids, n_out):
    N, C = h.shape
    NG = C // _LANES
    NR = _NW // NG
    Np = _cdiv(N, 16 * NR) * (16 * NR)
    if Np != N:
        h = jnp.pad(h, ((0, Np - N), (0, 0)))
        ids = jnp.pad(ids, (0, Np - N), constant_values=n_out)
    chunk = Np // NR

    mesh = plsc.VectorSubcoreMesh(core_axis_name="c", subcore_axis_name="s")

    @functools.partial(
        pl.kernel, mesh=mesh,
        out_type=jax.ShapeDtypeStruct((NR, n_out, C), jnp.float32),
        scratch_types=[
            pltpu.VMEM((chunk,), jnp.int32),
            pltpu.VMEM((chunk, _LANES), jnp.float32),
            pltpu.VMEM((n_out + 1, _LANES), jnp.float32),
            pltpu.SemaphoreType.DMA,
        ],
        compiler_params=pltpu.CompilerParams(use_tc_tiling_on_sc=False),
    )
    def k(h_h, ids_h, out_h, idsv, rowsv, accv, sem):
        cid = lax.axis_index("c")
        sid = lax.axis_index("s")
        wid = sid * _NC + cid
        g = wid % NG
        r = wid // NG
        cp_i = pltpu.async_copy(ids_h.at[pl.ds(r * chunk, chunk)], idsv, sem)
        cp_r = pltpu.async_copy(
            h_h.at[pl.ds(r * chunk, chunk), pl.ds(g * _LANES, _LANES)],
            rowsv, sem)
        zero = jnp.zeros((_LANES,), jnp.float32)

        def zbody(i, carry):
            accv[i, :] = zero
            return carry

        lax.fori_loop(0, n_out + 1, zbody, 0)
        cp_i.wait()
        cp_r.wait()

        def body(kk, carry):
            base = kk * 16
            ids16 = idsv[pl.ds(base, 16)]
            for j in range(16):
                idj = ids16[j]
                accv[idj, :] = jnp.maximum(accv[idj, :], rowsv[base + j, :])
            return carry

        lax.fori_loop(0, chunk // 16, body, 0)
        pltpu.sync_copy(accv.at[pl.ds(0, n_out)],
                        out_h.at[r, :, pl.ds(g * _LANES, _LANES)])

    return k(h, ids)


def _pool_reduce_body(part_ref, o_ref):
    o_ref[...] = jnp.max(part_ref[...], axis=0)


def _pool(h, ids, n_out):
    part = _pool_sc(h, ids, n_out)
    return pl.pallas_call(
        _pool_reduce_body,
        out_shape=jax.ShapeDtypeStruct((n_out, h.shape[1]), jnp.float32),
    )(part)


def kernel(x, edge_index0, edge_index1, edge_index2, edge_index3, edge_index4,
           pool1_ids, pool2_ids, pool3_ids, pool4_ids,
           Wn1, Ws1, g1, b1, Wn2, Ws2, g2, b2, Wn3, Ws3, g3, b3,
           Wn4, Ws4, g4, b4, Wn5, Ws5, g5, b5, Wn6, Ws6, g6, b6,
           Wn7, Ws7, g7, b7, Wn8, Ws8, g8, b8, Wn9, Ws9, g9, b9,
           Wn10, Ws10, g10, b10, Wn11, Ws11, g11, b11,
           Wn12, Ws12, g12, b12, Wn13, Ws13, g13, b13):
    p = locals()
    e = [p['edge_index%d' % i] for i in range(5)]

    def layer(h, i, ei):
        aggp = _segsum_sc(h, ei[0], ei[1], h.shape[0])
        return _layer(aggp, h, p['Wn%d' % i], p['Ws%d' % i],
                      p['g%d' % i], p['b%d' % i])

    h = layer(x, 1, e[0])
    h = layer(h, 2, e[0])
    h = _pool(h, pool1_ids, _NODES[1])
    h = layer(h, 3, e[1])
    out2 = layer(h, 4, e[1])
    h = _pool(out2, pool2_ids, _NODES[2])
    h = layer(h, 5, e[2])
    h = layer(h, 6, e[2])
    h = layer(h, 7, e[2])
    out3 = _pool(h, pool3_ids, _NODES[3])
    h = layer(out3, 8, e[3])
    h = layer(h, 9, e[3])
    h = layer(h, 10, e[3])
    h = _pool(h, pool4_ids, _NODES[4])
    h = layer(h, 11, e[4])
    h = layer(h, 12, e[4])
    out4 = layer(h, 13, e[4])
    return _embed(out2, out3, out4)


# speedup vs baseline: 1.0020x; 1.0020x over previous
import functools

import jax
import jax.numpy as jnp
from jax import lax
from jax.experimental import pallas as pl
from jax.experimental.pallas import tpu as pltpu
from jax.experimental.pallas import tpu_sc as plsc

_NODES = [10000, 2500, 625, 160, 40]

_NC, _NS, _LANES = 2, 16, 16
_NW = _NC * _NS
_K = 128


def _cdiv(a, b):
    return -(-a // b)


def _segsum_sc(h, src, dst, n_nodes):
    C = h.shape[1]
    E = src.shape[0]
    ZR = max(8, min(32, 8192 // C))
    Npad = _cdiv(n_nodes + 1, _NS * ZR) * (_NS * ZR)
    slab = Npad // _NS
    for K in (128, 96, 64, 32):
        Epad = _cdiv(E, K * _NW * 2) * (K * _NW * 2)
        cpw = Epad // (K * _NW)
        per_tile = 2 * cpw * K + 2 * K * C + ZR * C
        if per_tile <= 126000 and _NS * per_tile + Npad * C <= 2000000:
            break
    if Epad != E:
        pad = Epad - E
        src = jnp.concatenate([src, jnp.arange(pad, dtype=jnp.int32) % n_nodes])
        dst = jnp.concatenate([
            dst, n_nodes + jnp.arange(pad, dtype=jnp.int32) % (Npad - n_nodes)])
    src2 = src.reshape(-1, K)
    dst2 = dst.reshape(-1, K)

    mesh = plsc.VectorSubcoreMesh(core_axis_name="c", subcore_axis_name="s")

    @functools.partial(
        pl.kernel, mesh=mesh,
        out_type=jax.ShapeDtypeStruct((_NC, Npad, C), jnp.float32),
        scratch_types=[
            pltpu.VMEM((cpw, K), jnp.int32),
            pltpu.VMEM((cpw, K), jnp.int32),
            pltpu.VMEM((2, K, C), jnp.float32),
            pltpu.VMEM((ZR, C), jnp.float32),
            pltpu.VMEM_SHARED((Npad, C), jnp.float32),
            pltpu.SemaphoreType.DMA,
            pltpu.SemaphoreType.DMA,
            pltpu.SemaphoreType.DMA,
        ],
        compiler_params=pltpu.CompilerParams(use_tc_tiling_on_sc=False),
    )
    def k(src_h, dst_h, h_h, out_h, idx_s, idx_d, rows, zrow, acc, si, g0, g1):
        cid = lax.axis_index("c")
        sid = lax.axis_index("s")
        wid = sid * _NC + cid
        c0 = wid * cpw
        cp_s = pltpu.async_copy(src_h.at[pl.ds(c0, cpw)], idx_s, si)
        cp_d = pltpu.async_copy(dst_h.at[pl.ds(c0, cpw)], idx_d, si)

        zero = jnp.zeros((_LANES,), jnp.float32)
        for r in range(ZR):
            for j in range(C // _LANES):
                zrow[r, pl.ds(j * _LANES, _LANES)] = zero

        @pl.when(sid * slab < n_nodes)
        def _():
            for t in range(slab // ZR):
                pltpu.sync_copy(zrow, acc.at[pl.ds(sid * slab + t * ZR, ZR)])

        cp_s.wait()
        cp_d.wait()
        plsc.subcore_barrier()

        pltpu.async_copy(h_h.at[idx_s.at[0]], rows.at[0], g0)

        def body(it, carry):
            a = 2 * it
            b = a + 1
            pltpu.async_copy(h_h.at[idx_s.at[b]], rows.at[1], g1)
            pltpu.make_async_copy(h_h.at[idx_s.at[a]], rows.at[0], g0).wait()
            pltpu.sync_copy(rows.at[0], acc.at[idx_d.at[a]], add=True)

            @pl.when(b + 1 < cpw)
            def _():
                pltpu.async_copy(h_h.at[idx_s.at[b + 1]], rows.at[0], g0)

            pltpu.make_async_copy(h_h.at[idx_s.at[b]], rows.at[1], g1).wait()
            pltpu.sync_copy(rows.at[1], acc.at[idx_d.at[b]], add=True)
            return carry

        lax.fori_loop(0, cpw // 2, body, 0)
        plsc.subcore_barrier()

        @pl.when(sid * slab < n_nodes)
        def _():
            pltpu.sync_copy(acc.at[pl.ds(sid * slab, slab)],
                            out_h.at[cid, pl.ds(sid * slab, slab)])

    return k(src2, dst2, h)


def _layer_body(n, c_in, aggp_ref, h_ref, wn_ref, ws_ref, g_ref, b_ref, o_ref):
    agg = aggp_ref[0, :n, :c_in] + aggp_ref[1, :n, :c_in]
    z = jnp.dot(agg, wn_ref[...], preferred_element_type=jnp.float32)
    z = z + jnp.dot(h_ref[...], ws_ref[...], preferred_element_type=jnp.float32)
    mu = jnp.mean(z, axis=0, keepdims=True)
    zc = z - mu
    var = jnp.mean(zc * zc, axis=0, keepdims=True)
    zn = zc * lax.rsqrt(var + 1e-5) * g_ref[...] + b_ref[...]
    o_ref[...] = jnp.maximum(zn, 0.0)


def _layer(aggp, h, wn, ws, g, b):
    n, cout = h.shape[0], wn.shape[1]
    return pl.pallas_call(
        functools.partial(_layer_body, n, wn.shape[0]),
        out_shape=jax.ShapeDtypeStruct((n, cout), jnp.float32),
    )(aggp, h, wn, ws, g.reshape(1, -1), b.reshape(1, -1))


def _emb_body(o2_ref, o3_ref, o4_ref, out_ref):
    m2 = jnp.mean(o2_ref[...], axis=0, keepdims=True)
    m3 = jnp.mean(o3_ref[...], axis=0, keepdims=True)
    m4 = jnp.mean(o4_ref[...], axis=0, keepdims=True)
    ss = jnp.sum(m2 * m2) + jnp.sum(m3 * m3) + jnp.sum(m4 * m4)
    inv = 1.0 / jnp.maximum(jnp.sqrt(ss), 1e-12)
    out_ref[0:1, 0:128] = m2 * inv
    out_ref[0:1, 128:384] = m3 * inv
    out_ref[0:1, 384:896] = m4 * inv


def _embed(o2, o3, o4):
    return pl.pallas_call(
        _emb_body,
        out_shape=jax.ShapeDtypeStruct((1, 896), jnp.float32),
    )(o2, o3, o4)


def _pool_sc(h, ids, n_out):
    N, C = h.shape
    NG = C // _LANES
    NR = _NW // NG
    Np = _cdiv(N, 16 * NR) * (16 * NR)
    if Np != N:
        h = jnp.pad(h, ((0, Np - N), (0, 0)))
        ids = jnp.pad(ids, (0, Np - N), constant_values=n_out)
    chunk = Np // NR

    mesh = plsc.VectorSubcoreMesh(core_axis_name="c", subcore_axis_name="s")

    @functools.partial(
        pl.kernel, mesh=mesh,
        out_type=jax.ShapeDtypeStruct((NR, n_out, C), jnp.float32),
        scratch_types=[
            pltpu.VMEM((chunk,), jnp.int32),
            pltpu.VMEM((chunk, _LANES), jnp.float32),
            pltpu.VMEM((n_out + 1, _LANES), jnp.float32),
            pltpu.SemaphoreType.DMA,
        ],
        compiler_params=pltpu.CompilerParams(use_tc_tiling_on_sc=False),
    )
    def k(h_h, ids_h, out_h, idsv, rowsv, accv, sem):
        cid = lax.axis_index("c")
        sid = lax.axis_index("s")
        wid = sid * _NC + cid
        g = wid % NG
        r = wid // NG
        cp_i = pltpu.async_copy(ids_h.at[pl.ds(r * chunk, chunk)], idsv, sem)
        cp_r = pltpu.async_copy(
            h_h.at[pl.ds(r * chunk, chunk), pl.ds(g * _LANES, _LANES)],
            rowsv, sem)
        zero = jnp.zeros((_LANES,), jnp.float32)

        def zbody(i, carry):
            accv[i, :] = zero
            return carry

        lax.fori_loop(0, n_out + 1, zbody, 0)
        cp_i.wait()
        cp_r.wait()

        def body(kk, carry):
            base = kk * 16
            ids16 = idsv[pl.ds(base, 16)]
            for j in range(16):
                idj = ids16[j]
                accv[idj, :] = jnp.maximum(accv[idj, :], rowsv[base + j, :])
            return carry

        lax.fori_loop(0, chunk // 16, body, 0)
        pltpu.sync_copy(accv.at[pl.ds(0, n_out)],
                        out_h.at[r, :, pl.ds(g * _LANES, _LANES)])

    return k(h, ids)


def _pool_reduce_body(part_ref, o_ref):
    o_ref[...] = jnp.max(part_ref[...], axis=0)


def _pool(h, ids, n_out):
    part = _pool_sc(h, ids, n_out)
    return pl.pallas_call(
        _pool_reduce_body,
        out_shape=jax.ShapeDtypeStruct((n_out, h.shape[1]), jnp.float32),
    )(part)


def kernel(x, edge_index0, edge_index1, edge_index2, edge_index3, edge_index4,
           pool1_ids, pool2_ids, pool3_ids, pool4_ids,
           Wn1, Ws1, g1, b1, Wn2, Ws2, g2, b2, Wn3, Ws3, g3, b3,
           Wn4, Ws4, g4, b4, Wn5, Ws5, g5, b5, Wn6, Ws6, g6, b6,
           Wn7, Ws7, g7, b7, Wn8, Ws8, g8, b8, Wn9, Ws9, g9, b9,
           Wn10, Ws10, g10, b10, Wn11, Ws11, g11, b11,
           Wn12, Ws12, g12, b12, Wn13, Ws13, g13, b13):
    p = locals()
    e = [p['edge_index%d' % i] for i in range(5)]

    def layer(h, i, ei):
        aggp = _segsum_sc(h, ei[0], ei[1], h.shape[0])
        return _layer(aggp, h, p['Wn%d' % i], p['Ws%d' % i],
                      p['g%d' % i], p['b%d' % i])

    h = layer(x, 1, e[0])
    h = layer(h, 2, e[0])
    h = _pool(h, pool1_ids, _NODES[1])
    h = layer(h, 3, e[1])
    out2 = layer(h, 4, e[1])
    h = _pool(out2, pool2_ids, _NODES[2])
    h = layer(h, 5, e[2])
    h = layer(h, 6, e[2])
    h = layer(h, 7, e[2])
    out3 = _pool(h, pool3_ids, _NODES[3])
    h = layer(out3, 8, e[3])
    h = layer(h, 9, e[3])
    h = layer(h, 10, e[3])
    h = _pool(h, pool4_ids, _NODES[4])
    h = layer(h, 11, e[4])
    h = layer(h, 12, e[4])
    out4 = layer(h, 13, e[4])
    return _embed(out2, out3, out4)
